# baseline (speedup 1.0000x reference)
import functools

import jax
import jax.numpy as jnp
from jax import lax
from jax.experimental import pallas as pl
from jax.experimental.pallas import tpu as pltpu
from jax.experimental.pallas import tpu_sc as plsc

N = 10000
E = 320000
HID = 128


_NC = 2
_NS = 16
_NW = _NC * _NS
_BLK = 128
_NBLK = E // _BLK
_SITER = -(-_NBLK // _NS)
_RPS = 632
_RPS_LAST = N - 15 * _RPS

_GBLK = 128
_GNBLK = E // _GBLK
_GITER = -(-_GNBLK // _NW)


def _sc_fuse_gather(tsrc, tdst, ce, src, dst):
    mesh = plsc.VectorSubcoreMesh(core_axis_name="c", subcore_axis_name="s")

    @functools.partial(
        pl.kernel,
        out_type=jax.ShapeDtypeStruct((E, 256), jnp.float32),
        mesh=mesh,
        scratch_types=[
            pltpu.VMEM((_GBLK,), jnp.int32), pltpu.VMEM((_GBLK,), jnp.int32),
            pltpu.VMEM((_GBLK,), jnp.int32), pltpu.VMEM((_GBLK,), jnp.int32),
            pltpu.VMEM((_GBLK, 256), jnp.float32),
            pltpu.VMEM((_GBLK, 256), jnp.float32),
            pltpu.VMEM((_GBLK, 128), jnp.float32),
            pltpu.VMEM((_GBLK, 128), jnp.float32),
            pltpu.VMEM((_GBLK, 128), jnp.float32),
            pltpu.SemaphoreType.DMA, pltpu.SemaphoreType.DMA,
            pltpu.SemaphoreType.DMA, pltpu.SemaphoreType.DMA,
            pltpu.SemaphoreType.DMA, pltpu.SemaphoreType.DMA,
        ],
    )
    def k(ts_hbm, td_hbm, ce_hbm, src_hbm, dst_hbm, out_hbm,
          si0, si1, di0, di1, A0, A1, E0, E1, C,
          sA0, sA1, sE0, sE1, sO0, sO1):
        wid = lax.axis_index("s") * _NC + lax.axis_index("c")

        def issue(b, si, di, A, Ebuf, sA, sE):
            base = b * _GBLK
            pltpu.sync_copy(src_hbm.at[pl.ds(base, _GBLK)], si)
            pltpu.sync_copy(dst_hbm.at[pl.ds(base, _GBLK)], di)
            pltpu.async_copy(ts_hbm.at[si], A, sA)
            pltpu.async_copy(td_hbm.at[di], Ebuf, sE)

        def half(t, cur, nxt):
            (csi, cdi, cA, cE, csA, csE, csO) = cur
            (nsi, ndi, nA, nE, nsA, nsE, nsO) = nxt
            b = t * _NW + wid
            bn = b + _NW

            @pl.when(bn < _GNBLK)
            def _nxt():
                @pl.when(t >= 1)
                def _wo():
                    pltpu.make_async_copy(
                        nA, out_hbm.at[pl.ds(0, _GBLK)], nsO).wait()
                issue(bn, nsi, ndi, nA, nE, nsA, nsE)

            @pl.when(b < _GNBLK)
            def _cur():
                pltpu.sync_copy(ce_hbm.at[pl.ds(b * _GBLK, _GBLK)], C)
                pltpu.make_async_copy(ts_hbm.at[csi], cA, csA).wait()
                pltpu.make_async_copy(td_hbm.at[cdi], cE, csE).wait()

                def rbody(r, carry):
                    for j in range(8):
                        sl = pl.ds(j * 16, 16)
                        cA[r, sl] = cA[r, sl] + cE[r, sl] + C[r, sl]
                    return carry

                lax.fori_loop(0, _GBLK, rbody, 0)
                pltpu.async_copy(cA, out_hbm.at[pl.ds(b * _GBLK, _GBLK)], csO)

        bufs0 = (si0, di0, A0, E0, sA0, sE0, sO0)
        bufs1 = (si1, di1, A1, E1, sA1, sE1, sO1)
        issue(wid, si0, di0, A0, E0, sA0, sE0)

        def body(t, carry):
            @pl.when(t % 2 == 0)
            def _e():
                half(t, bufs0, bufs1)

            @pl.when(t % 2 == 1)
            def _o():
                half(t, bufs1, bufs0)
            return carry

        lax.fori_loop(0, _GITER, body, 0)
        pltpu.make_async_copy(A0, out_hbm.at[pl.ds(0, _GBLK)], sO0).wait()
        pltpu.make_async_copy(A1, out_hbm.at[pl.ds(0, _GBLK)], sO1).wait()

    return k(tsrc, tdst, ce, src, dst)


def _sc_scatter(msg, sig, dst, zeros):
    mesh = plsc.VectorSubcoreMesh(core_axis_name="c", subcore_axis_name="s")

    @functools.partial(
        pl.kernel,
        out_type=[
            jax.ShapeDtypeStruct((N, 128), jnp.float32),
            jax.ShapeDtypeStruct((N, 128), jnp.float32),
        ],
        mesh=mesh,
        scratch_types=[
            pltpu.VMEM((_BLK,), jnp.int32),
            pltpu.VMEM((_BLK,), jnp.int32),
            pltpu.VMEM((_BLK, 128), jnp.float32),
            pltpu.VMEM((_BLK, 128), jnp.float32),
            pltpu.VMEM_SHARED((N, 128), jnp.float32),
            pltpu.SemaphoreType.DMA, pltpu.SemaphoreType.DMA,
            pltpu.SemaphoreType.DMA, pltpu.SemaphoreType.DMA,
        ],
    )
    def k(msg_hbm, sig_hbm, dst_hbm, z_hbm, num_hbm, den_hbm,
          idx0, idx1, upd0, upd1, acc, sI0, sI1, sU0, sU1):
        c = lax.axis_index("c")
        s = lax.axis_index("s")

        @pl.when(s < 15)
        def _z0():
            pltpu.sync_copy(z_hbm, acc.at[pl.ds(s * _RPS, _RPS)])

        @pl.when(s == 15)
        def _z1():
            pltpu.sync_copy(z_hbm.at[pl.ds(0, _RPS_LAST)],
                            acc.at[pl.ds(15 * _RPS, _RPS_LAST)])

        plsc.subcore_barrier()

        def issue(b, idx_v, upd_v, sI, sU):
            base = b * _BLK
            pltpu.async_copy(dst_hbm.at[pl.ds(base, _BLK)], idx_v, sI)

            @pl.when(c == 0)
            def _c0():
                pltpu.async_copy(msg_hbm.at[pl.ds(base, _BLK)], upd_v, sU)

            @pl.when(c == 1)
            def _c1():
                pltpu.async_copy(sig_hbm.at[pl.ds(base, _BLK)], upd_v, sU)

        def half(t, cur, nxt):
            (cidx, cupd, csI, csU) = cur
            (nidx, nupd, nsI, nsU) = nxt
            b = t * _NS + s
            bn = b + _NS

            @pl.when(bn < _NBLK)
            def _nxt():
                issue(bn, nidx, nupd, nsI, nsU)

            @pl.when(b < _NBLK)
            def _cur():
                pltpu.make_async_copy(
                    dst_hbm.at[pl.ds(0, _BLK)], cidx, csI).wait()
                pltpu.make_async_copy(
                    msg_hbm.at[pl.ds(0, _BLK)], cupd, csU).wait()
                pltpu.sync_copy(cupd, acc.at[cidx], add=True)

        bufs0 = (idx0, upd0, sI0, sU0)
        bufs1 = (idx1, upd1, sI1, sU1)
        issue(s, idx0, upd0, sI0, sU0)

        def body(t, carry):
            @pl.when(t % 2 == 0)
            def _e():
                half(t, bufs0, bufs1)

            @pl.when(t % 2 == 1)
            def _o():
                half(t, bufs1, bufs0)
            return carry

        lax.fori_loop(0, _SITER, body, 0)
        plsc.subcore_barrier()

        @pl.when(jnp.logical_and(c == 0, s < 15))
        def _w00():
            pltpu.sync_copy(acc.at[pl.ds(s * _RPS, _RPS)],
                            num_hbm.at[pl.ds(s * _RPS, _RPS)])

        @pl.when(jnp.logical_and(c == 0, s == 15))
        def _w01():
            pltpu.sync_copy(acc.at[pl.ds(15 * _RPS, _RPS_LAST)],
                            num_hbm.at[pl.ds(15 * _RPS, _RPS_LAST)])

        @pl.when(jnp.logical_and(c == 1, s < 15))
        def _w10():
            pltpu.sync_copy(acc.at[pl.ds(s * _RPS, _RPS)],
                            den_hbm.at[pl.ds(s * _RPS, _RPS)])

        @pl.when(jnp.logical_and(c == 1, s == 15))
        def _w11():
            pltpu.sync_copy(acc.at[pl.ds(15 * _RPS, _RPS_LAST)],
                            den_hbm.at[pl.ds(15 * _RPS, _RPS_LAST)])

    return k(msg, sig, dst, zeros)


_NBK = 2000
_EBK = 1280


def _node_mm_body(h_ref, w_ref, b_ref, ah_ref, ts_ref, td_ref):
    x = h_ref[...] @ w_ref[...] + b_ref[...]
    ah_ref[...] = x[:, 0:128]
    ts_ref[...] = x[:, 128:384]
    td_ref[...] = x[:, 384:512]


def _node_mm(h, wcat, bcat):
    grid = (N // _NBK,)
    return pl.pallas_call(
        _node_mm_body,
        grid=grid,
        in_specs=[
            pl.BlockSpec((_NBK, HID), lambda i: (i, 0)),
            pl.BlockSpec((HID, 512), lambda i: (0, 0)),
            pl.BlockSpec((1, 512), lambda i: (0, 0)),
        ],
        out_specs=[
            pl.BlockSpec((_NBK, 128), lambda i: (i, 0)),
            pl.BlockSpec((_NBK, 256), lambda i: (i, 0)),
            pl.BlockSpec((_NBK, 128), lambda i: (i, 0)),
        ],
        out_shape=[
            jax.ShapeDtypeStruct((N, 128), jnp.float32),
            jax.ShapeDtypeStruct((N, 256), jnp.float32),
            jax.ShapeDtypeStruct((N, 128), jnp.float32),
        ],
    )(h, wcat, bcat)


def _edge0_body(ef_ref, we_ref, be_ref, wc_ref, bc_ref, e_ref, ce_ref):
    e = ef_ref[...] @ we_ref[...] + be_ref[...]
    e_ref[...] = e
    ce_ref[...] = e @ wc_ref[...] + bc_ref[...]


def _edge0(edge_feat, We, be, wc, bc):
    grid = (E // _EBK,)
    d_in = edge_feat.shape[1]
    return pl.pallas_call(
        _edge0_body,
        grid=grid,
        in_specs=[
            pl.BlockSpec((_EBK, d_in), lambda i: (i, 0)),
            pl.BlockSpec((d_in, HID), lambda i: (0, 0)),
            pl.BlockSpec((1, HID), lambda i: (0, 0)),
            pl.BlockSpec((HID, HID), lambda i: (0, 0)),
            pl.BlockSpec((1, HID), lambda i: (0, 0)),
        ],
        out_specs=[
            pl.BlockSpec((_EBK, HID), lambda i: (i, 0)),
            pl.BlockSpec((_EBK, HID), lambda i: (i, 0)),
        ],
        out_shape=[
            jax.ShapeDtypeStruct((E, HID), jnp.float32),
            jax.ShapeDtypeStruct((E, HID), jnp.float32),
        ],
    )(edge_feat, We, be, wc, bc)


def _eupd_body(ep_ref, en_ref, mu_ref, iv_ref, g_ref, b_ref,
               wc_ref, bc_ref, e_ref, ce_ref):
    bn = (en_ref[...] - mu_ref[...]) * iv_ref[...] * g_ref[...] + b_ref[...]
    e = ep_ref[...] + jnp.maximum(bn, 0.0)
    e_ref[...] = e
    ce_ref[...] = e @ wc_ref[...] + bc_ref[...]


def _edge_update(e_prev, e_new_prev, mu, ivar, g, b, wc, bc):
    grid = (E // _EBK,)
    row = lambda: pl.BlockSpec((1, HID), lambda i: (0, 0))
    return pl.pallas_call(
        _eupd_body,
        grid=grid,
        in_specs=[
            pl.BlockSpec((_EBK, HID), lambda i: (i, 0)),
            pl.BlockSpec((_EBK, HID), lambda i: (i, 0)),
            row(), row(), row(), row(),
            pl.BlockSpec((HID, HID), lambda i: (0, 0)),
            row(),
        ],
        out_specs=[
            pl.BlockSpec((_EBK, HID), lambda i: (i, 0)),
            pl.BlockSpec((_EBK, HID), lambda i: (i, 0)),
        ],
        out_shape=[
            jax.ShapeDtypeStruct((E, HID), jnp.float32),
            jax.ShapeDtypeStruct((E, HID), jnp.float32),
        ],
    )(e_prev, e_new_prev, mu, ivar, g, b, wc, bc)


def _gate_body(g_ref, sig_ref, msg_ref, s1_ref, s2_ref):
    i = pl.program_id(0)
    g = g_ref[...]
    e_new = g[:, 0:128]
    sig = jax.nn.sigmoid(e_new)
    sig_ref[...] = sig
    msg_ref[...] = g[:, 128:256] * sig
    ps = jnp.sum(e_new, axis=0, keepdims=True)
    pq = jnp.sum(e_new * e_new, axis=0, keepdims=True)

    @pl.when(i == 0)
    def _():
        s1_ref[...] = jnp.zeros_like(s1_ref)
        s2_ref[...] = jnp.zeros_like(s2_ref)

    s1_ref[0:1, :] += ps
    s2_ref[0:1, :] += pq


def _gate(g):
    grid = (E // _EBK,)
    return pl.pallas_call(
        _gate_body,
        grid=grid,
        in_specs=[
            pl.BlockSpec((_EBK, 256), lambda i: (i, 0)),
        ],
        out_specs=[
            pl.BlockSpec((_EBK, 128), lambda i: (i, 0)),
            pl.BlockSpec((_EBK, 128), lambda i: (i, 0)),
            pl.BlockSpec((8, 128), lambda i: (0, 0)),
            pl.BlockSpec((8, 128), lambda i: (0, 0)),
        ],
        out_shape=[
            jax.ShapeDtypeStruct((E, 128), jnp.float32),
            jax.ShapeDtypeStruct((E, 128), jnp.float32),
            jax.ShapeDtypeStruct((8, 128), jnp.float32),
            jax.ShapeDtypeStruct((8, 128), jnp.float32),
        ],
    )(g)


def _hupd_body(hin_ref, ah_ref, num_ref, den_ref, g_ref, b_ref, out_ref):
    h_new = ah_ref[...] + num_ref[...] / (den_ref[...] + 1e-6)
    mu = jnp.mean(h_new, axis=0, keepdims=True)
    var = jnp.mean((h_new - mu) ** 2, axis=0, keepdims=True)
    bn = (h_new - mu) * jax.lax.rsqrt(var + 1e-5) * g_ref[...] + b_ref[...]
    out_ref[...] = hin_ref[...] + jnp.maximum(bn, 0.0)


def _h_update(h_in, ah, num, den, g, b):
    full = lambda s: pl.BlockSpec(s, lambda: tuple(0 for _ in s))
    return pl.pallas_call(
        _hupd_body,
        in_specs=[
            full((N, HID)), full((N, HID)), full((N, HID)), full((N, HID)),
            full((1, HID)), full((1, HID)),
        ],
        out_specs=full((N, HID)),
        out_shape=jax.ShapeDtypeStruct((N, HID), jnp.float32),
    )(h_in, ah, num, den, g, b)


def _mlp_body(h_ref, w1, b1, w2, b2, w3, b3, o_ref):
    y = h_ref[...]
    y = jnp.maximum(y @ w1[...] + b1[...], 0.0)
    y = jnp.maximum(y @ w2[...] + b2[...], 0.0)
    o_ref[...] = y @ w3[...] + b3[...]


def _mlp_readout(h, mlp):
    blk = 2000
    grid = (N // blk,)
    w1, b1 = mlp[0]["W"], mlp[0]["b"].reshape(1, -1)
    w2, b2 = mlp[1]["W"], mlp[1]["b"].reshape(1, -1)
    w3, b3 = mlp[2]["W"], mlp[2]["b"].reshape(1, -1)
    full = lambda s: pl.BlockSpec(s, lambda i: (0,) * len(s))
    return pl.pallas_call(
        _mlp_body,
        grid=grid,
        in_specs=[
            pl.BlockSpec((blk, HID), lambda i: (i, 0)),
            full(w1.shape), full(b1.shape),
            full(w2.shape), full(b2.shape),
            full(w3.shape), full(b3.shape),
        ],
        out_specs=pl.BlockSpec((blk, w3.shape[1]), lambda i: (i, 0)),
        out_shape=jax.ShapeDtypeStruct((N, w3.shape[1]), jnp.float32),
    )(h, w1, b1, w2, b2, w3, b3)


def kernel(h, edge_index, edge_feat, We, be, layers, mlp):
    src = edge_index[0].astype(jnp.int32)
    dst = edge_index[1].astype(jnp.int32)
    zeros = jnp.zeros((_RPS, 128), jnp.float32)

    e_prev = None
    e_new_prev = None
    e_stats_prev = None
    for li, lp in enumerate(layers):
        wcat = jnp.concatenate(
            [lp["WA"], lp["WD"], lp["WB"], lp["WE"]], axis=1)
        bcat = jnp.concatenate(
            [lp["bA"], lp["bD"], lp["bB"], lp["bE"]], axis=0).reshape(1, 512)
        ah, tsrc, tdst = _node_mm(h, wcat, bcat)

        if li == 0:
            e_cur, ce = _edge0(edge_feat, We, be.reshape(1, HID),
                               lp["WC"], lp["bC"].reshape(1, HID))
        else:
            s1, s2 = e_stats_prev
            mu = (s1[0:1] / E)
            var = s2[0:1] / E - mu * mu
            ivar = jax.lax.rsqrt(var + 1e-5)
            lpp = layers[li - 1]
            e_cur, ce = _edge_update(
                e_prev, e_new_prev, mu, ivar,
                lpp["bn_e_g"].reshape(1, HID), lpp["bn_e_b"].reshape(1, HID),
                lp["WC"], lp["bC"].reshape(1, HID))

        g = _sc_fuse_gather(tsrc, tdst, ce, src, dst)
        sig, msg, s1, s2 = _gate(g)
        num, den = _sc_scatter(msg, sig, dst, zeros)
        e_new = g
        h = _h_update(h, ah, num, den,
                      lp["bn_h_g"].reshape(1, HID), lp["bn_h_b"].reshape(1, HID))
        e_prev, e_new_prev, e_stats_prev = e_cur, e_new, (s1, s2)

    return _mlp_readout(h, mlp)


# speedup vs baseline: 1.3723x; 1.3723x over previous
import functools

import jax
import jax.numpy as jnp
from jax import lax
from jax.experimental import pallas as pl
from jax.experimental.pallas import tpu as pltpu
from jax.experimental.pallas import tpu_sc as plsc

N = 10000
E = 320000
HID = 128


_NC = 2
_NS = 16
_NW = _NC * _NS
_BLK = 128
_NBLK = E // _BLK
_SITER = -(-_NBLK // _NS)
_RPS = 632
_RPS_LAST = N - 15 * _RPS

_GBLK = 128
_GNBLK = E // _GBLK
_GITER = -(-_GNBLK // _NW)


def _sc_gather2(tsrc, tdst, src, dst):
    mesh = plsc.VectorSubcoreMesh(core_axis_name="c", subcore_axis_name="s")

    @functools.partial(
        pl.kernel,
        out_type=[
            jax.ShapeDtypeStruct((E, 256), jnp.float32),
            jax.ShapeDtypeStruct((E, 128), jnp.float32),
        ],
        mesh=mesh,
        scratch_types=[
            pltpu.VMEM((_GBLK,), jnp.int32), pltpu.VMEM((_GBLK,), jnp.int32),
            pltpu.VMEM((_GBLK,), jnp.int32), pltpu.VMEM((_GBLK,), jnp.int32),
            pltpu.VMEM((_GBLK, 256), jnp.float32),
            pltpu.VMEM((_GBLK, 256), jnp.float32),
            pltpu.VMEM((_GBLK, 128), jnp.float32),
            pltpu.VMEM((_GBLK, 128), jnp.float32),
            pltpu.SemaphoreType.DMA, pltpu.SemaphoreType.DMA,
            pltpu.SemaphoreType.DMA, pltpu.SemaphoreType.DMA,
            pltpu.SemaphoreType.DMA, pltpu.SemaphoreType.DMA,
            pltpu.SemaphoreType.DMA, pltpu.SemaphoreType.DMA,
        ],
    )
    def k(ts_hbm, td_hbm, src_hbm, dst_hbm, osrc_hbm, odst_hbm,
          si0, si1, di0, di1, A0, A1, E0, E1,
          sA0, sA1, sE0, sE1, sO0, sO1, sP0, sP1):
        wid = lax.axis_index("s") * _NC + lax.axis_index("c")

        def issue(b, si, di, A, Ebuf, sA, sE):
            base = b * _GBLK
            pltpu.sync_copy(src_hbm.at[pl.ds(base, _GBLK)], si)
            pltpu.sync_copy(dst_hbm.at[pl.ds(base, _GBLK)], di)
            pltpu.async_copy(ts_hbm.at[si], A, sA)
            pltpu.async_copy(td_hbm.at[di], Ebuf, sE)

        def half(t, cur, nxt):
            (csi, cdi, cA, cE, csA, csE, csO, csP) = cur
            (nsi, ndi, nA, nE, nsA, nsE, nsO, nsP) = nxt
            b = t * _NW + wid
            bn = b + _NW

            @pl.when(bn < _GNBLK)
            def _nxt():
                @pl.when(t >= 1)
                def _wo():
                    pltpu.make_async_copy(
                        nA, osrc_hbm.at[pl.ds(0, _GBLK)], nsO).wait()
                    pltpu.make_async_copy(
                        nE, odst_hbm.at[pl.ds(0, _GBLK)], nsP).wait()
                issue(bn, nsi, ndi, nA, nE, nsA, nsE)

            @pl.when(b < _GNBLK)
            def _cur():
                base = b * _GBLK
                pltpu.make_async_copy(ts_hbm.at[csi], cA, csA).wait()
                pltpu.make_async_copy(td_hbm.at[cdi], cE, csE).wait()
                pltpu.async_copy(cA, osrc_hbm.at[pl.ds(base, _GBLK)], csO)
                pltpu.async_copy(cE, odst_hbm.at[pl.ds(base, _GBLK)], csP)

        bufs0 = (si0, di0, A0, E0, sA0, sE0, sO0, sP0)
        bufs1 = (si1, di1, A1, E1, sA1, sE1, sO1, sP1)
        issue(wid, si0, di0, A0, E0, sA0, sE0)

        def body(t, carry):
            @pl.when(t % 2 == 0)
            def _e():
                half(t, bufs0, bufs1)

            @pl.when(t % 2 == 1)
            def _o():
                half(t, bufs1, bufs0)
            return carry

        lax.fori_loop(0, _GITER, body, 0)
        pltpu.make_async_copy(A0, osrc_hbm.at[pl.ds(0, _GBLK)], sO0).wait()
        pltpu.make_async_copy(E0, odst_hbm.at[pl.ds(0, _GBLK)], sP0).wait()
        pltpu.make_async_copy(A1, osrc_hbm.at[pl.ds(0, _GBLK)], sO1).wait()
        pltpu.make_async_copy(E1, odst_hbm.at[pl.ds(0, _GBLK)], sP1).wait()

    return k(tsrc, tdst, src, dst)


def _sc_scatter(msg, sig, dst, zeros):
    mesh = plsc.VectorSubcoreMesh(core_axis_name="c", subcore_axis_name="s")

    @functools.partial(
        pl.kernel,
        out_type=[
            jax.ShapeDtypeStruct((N, 128), jnp.float32),
            jax.ShapeDtypeStruct((N, 128), jnp.float32),
        ],
        mesh=mesh,
        scratch_types=[
            pltpu.VMEM((_BLK,), jnp.int32),
            pltpu.VMEM((_BLK,), jnp.int32),
            pltpu.VMEM((_BLK, 128), jnp.float32),
            pltpu.VMEM((_BLK, 128), jnp.float32),
            pltpu.VMEM_SHARED((N, 128), jnp.float32),
            pltpu.SemaphoreType.DMA, pltpu.SemaphoreType.DMA,
            pltpu.SemaphoreType.DMA, pltpu.SemaphoreType.DMA,
        ],
    )
    def k(msg_hbm, sig_hbm, dst_hbm, z_hbm, num_hbm, den_hbm,
          idx0, idx1, upd0, upd1, acc, sI0, sI1, sU0, sU1):
        c = lax.axis_index("c")
        s = lax.axis_index("s")

        @pl.when(s < 15)
        def _z0():
            pltpu.sync_copy(z_hbm, acc.at[pl.ds(s * _RPS, _RPS)])

        @pl.when(s == 15)
        def _z1():
            pltpu.sync_copy(z_hbm.at[pl.ds(0, _RPS_LAST)],
                            acc.at[pl.ds(15 * _RPS, _RPS_LAST)])

        plsc.subcore_barrier()

        def issue(b, idx_v, upd_v, sI, sU):
            base = b * _BLK
            pltpu.async_copy(dst_hbm.at[pl.ds(base, _BLK)], idx_v, sI)

            @pl.when(c == 0)
            def _c0():
                pltpu.async_copy(msg_hbm.at[pl.ds(base, _BLK)], upd_v, sU)

            @pl.when(c == 1)
            def _c1():
                pltpu.async_copy(sig_hbm.at[pl.ds(base, _BLK)], upd_v, sU)

        def half(t, cur, nxt):
            (cidx, cupd, csI, csU) = cur
            (nidx, nupd, nsI, nsU) = nxt
            b = t * _NS + s
            bn = b + _NS

            @pl.when(bn < _NBLK)
            def _nxt():
                issue(bn, nidx, nupd, nsI, nsU)

            @pl.when(b < _NBLK)
            def _cur():
                pltpu.make_async_copy(
                    dst_hbm.at[pl.ds(0, _BLK)], cidx, csI).wait()
                pltpu.make_async_copy(
                    msg_hbm.at[pl.ds(0, _BLK)], cupd, csU).wait()
                pltpu.sync_copy(cupd, acc.at[cidx], add=True)

        bufs0 = (idx0, upd0, sI0, sU0)
        bufs1 = (idx1, upd1, sI1, sU1)
        issue(s, idx0, upd0, sI0, sU0)

        def body(t, carry):
            @pl.when(t % 2 == 0)
            def _e():
                half(t, bufs0, bufs1)

            @pl.when(t % 2 == 1)
            def _o():
                half(t, bufs1, bufs0)
            return carry

        lax.fori_loop(0, _SITER, body, 0)
        plsc.subcore_barrier()

        @pl.when(jnp.logical_and(c == 0, s < 15))
        def _w00():
            pltpu.sync_copy(acc.at[pl.ds(s * _RPS, _RPS)],
                            num_hbm.at[pl.ds(s * _RPS, _RPS)])

        @pl.when(jnp.logical_and(c == 0, s == 15))
        def _w01():
            pltpu.sync_copy(acc.at[pl.ds(15 * _RPS, _RPS_LAST)],
                            num_hbm.at[pl.ds(15 * _RPS, _RPS_LAST)])

        @pl.when(jnp.logical_and(c == 1, s < 15))
        def _w10():
            pltpu.sync_copy(acc.at[pl.ds(s * _RPS, _RPS)],
                            den_hbm.at[pl.ds(s * _RPS, _RPS)])

        @pl.when(jnp.logical_and(c == 1, s == 15))
        def _w11():
            pltpu.sync_copy(acc.at[pl.ds(15 * _RPS, _RPS_LAST)],
                            den_hbm.at[pl.ds(15 * _RPS, _RPS_LAST)])

    return k(msg, sig, dst, zeros)


_NBK = 2000
_EBK = 1280


def _node_mm_body(h_ref, w_ref, b_ref, ah_ref, ts_ref, td_ref):
    x = h_ref[...] @ w_ref[...] + b_ref[...]
    ah_ref[...] = x[:, 0:128]
    ts_ref[...] = x[:, 128:384]
    td_ref[...] = x[:, 384:512]


def _node_mm(h, wcat, bcat):
    grid = (N // _NBK,)
    return pl.pallas_call(
        _node_mm_body,
        grid=grid,
        in_specs=[
            pl.BlockSpec((_NBK, HID), lambda i: (i, 0)),
            pl.BlockSpec((HID, 512), lambda i: (0, 0)),
            pl.BlockSpec((1, 512), lambda i: (0, 0)),
        ],
        out_specs=[
            pl.BlockSpec((_NBK, 128), lambda i: (i, 0)),
            pl.BlockSpec((_NBK, 256), lambda i: (i, 0)),
            pl.BlockSpec((_NBK, 128), lambda i: (i, 0)),
        ],
        out_shape=[
            jax.ShapeDtypeStruct((N, 128), jnp.float32),
            jax.ShapeDtypeStruct((N, 256), jnp.float32),
            jax.ShapeDtypeStruct((N, 128), jnp.float32),
        ],
    )(h, wcat, bcat)


def _edge0_body(ef_ref, we_ref, be_ref, wc_ref, bc_ref, e_ref, ce_ref):
    e = ef_ref[...] @ we_ref[...] + be_ref[...]
    e_ref[...] = e
    ce_ref[...] = e @ wc_ref[...] + bc_ref[...]


def _edge0(edge_feat, We, be, wc, bc):
    grid = (E // _EBK,)
    d_in = edge_feat.shape[1]
    return pl.pallas_call(
        _edge0_body,
        grid=grid,
        in_specs=[
            pl.BlockSpec((_EBK, d_in), lambda i: (i, 0)),
            pl.BlockSpec((d_in, HID), lambda i: (0, 0)),
            pl.BlockSpec((1, HID), lambda i: (0, 0)),
            pl.BlockSpec((HID, HID), lambda i: (0, 0)),
            pl.BlockSpec((1, HID), lambda i: (0, 0)),
        ],
        out_specs=[
            pl.BlockSpec((_EBK, HID), lambda i: (i, 0)),
            pl.BlockSpec((_EBK, HID), lambda i: (i, 0)),
        ],
        out_shape=[
            jax.ShapeDtypeStruct((E, HID), jnp.float32),
            jax.ShapeDtypeStruct((E, HID), jnp.float32),
        ],
    )(edge_feat, We, be, wc, bc)


def _eupd_body(ep_ref, en_ref, mu_ref, iv_ref, g_ref, b_ref,
               wc_ref, bc_ref, e_ref, ce_ref):
    bn = (en_ref[...] - mu_ref[...]) * iv_ref[...] * g_ref[...] + b_ref[...]
    e = ep_ref[...] + jnp.maximum(bn, 0.0)
    e_ref[...] = e
    ce_ref[...] = e @ wc_ref[...] + bc_ref[...]


def _edge_update(e_prev, e_new_prev, mu, ivar, g, b, wc, bc):
    grid = (E // _EBK,)
    row = lambda: pl.BlockSpec((1, HID), lambda i: (0, 0))
    return pl.pallas_call(
        _eupd_body,
        grid=grid,
        in_specs=[
            pl.BlockSpec((_EBK, HID), lambda i: (i, 0)),
            pl.BlockSpec((_EBK, HID), lambda i: (i, 0)),
            row(), row(), row(), row(),
            pl.BlockSpec((HID, HID), lambda i: (0, 0)),
            row(),
        ],
        out_specs=[
            pl.BlockSpec((_EBK, HID), lambda i: (i, 0)),
            pl.BlockSpec((_EBK, HID), lambda i: (i, 0)),
        ],
        out_shape=[
            jax.ShapeDtypeStruct((E, HID), jnp.float32),
            jax.ShapeDtypeStruct((E, HID), jnp.float32),
        ],
    )(e_prev, e_new_prev, mu, ivar, g, b, wc, bc)


def _gate_body(rs_ref, rd_ref, ce_ref, en_ref, sig_ref, msg_ref,
               s1_ref, s2_ref):
    i = pl.program_id(0)
    rs = rs_ref[...]
    e_new = rs[:, 0:128] + rd_ref[...] + ce_ref[...]
    sig = jax.nn.sigmoid(e_new)
    en_ref[...] = e_new
    sig_ref[...] = sig
    msg_ref[...] = rs[:, 128:256] * sig
    ps = jnp.sum(e_new, axis=0, keepdims=True)
    pq = jnp.sum(e_new * e_new, axis=0, keepdims=True)

    @pl.when(i == 0)
    def _():
        s1_ref[...] = jnp.zeros_like(s1_ref)
        s2_ref[...] = jnp.zeros_like(s2_ref)

    s1_ref[0:1, :] += ps
    s2_ref[0:1, :] += pq


def _gate(rows_src, rows_dst, ce):
    grid = (E // _EBK,)
    return pl.pallas_call(
        _gate_body,
        grid=grid,
        in_specs=[
            pl.BlockSpec((_EBK, 256), lambda i: (i, 0)),
            pl.BlockSpec((_EBK, 128), lambda i: (i, 0)),
            pl.BlockSpec((_EBK, 128), lambda i: (i, 0)),
        ],
        out_specs=[
            pl.BlockSpec((_EBK, 128), lambda i: (i, 0)),
            pl.BlockSpec((_EBK, 128), lambda i: (i, 0)),
            pl.BlockSpec((_EBK, 128), lambda i: (i, 0)),
            pl.BlockSpec((8, 128), lambda i: (0, 0)),
            pl.BlockSpec((8, 128), lambda i: (0, 0)),
        ],
        out_shape=[
            jax.ShapeDtypeStruct((E, 128), jnp.float32),
            jax.ShapeDtypeStruct((E, 128), jnp.float32),
            jax.ShapeDtypeStruct((E, 128), jnp.float32),
            jax.ShapeDtypeStruct((8, 128), jnp.float32),
            jax.ShapeDtypeStruct((8, 128), jnp.float32),
        ],
    )(rows_src, rows_dst, ce)


def _hupd_body(hin_ref, ah_ref, num_ref, den_ref, g_ref, b_ref, out_ref):
    h_new = ah_ref[...] + num_ref[...] / (den_ref[...] + 1e-6)
    mu = jnp.mean(h_new, axis=0, keepdims=True)
    var = jnp.mean((h_new - mu) ** 2, axis=0, keepdims=True)
    bn = (h_new - mu) * jax.lax.rsqrt(var + 1e-5) * g_ref[...] + b_ref[...]
    out_ref[...] = hin_ref[...] + jnp.maximum(bn, 0.0)


def _h_update(h_in, ah, num, den, g, b):
    full = lambda s: pl.BlockSpec(s, lambda: tuple(0 for _ in s))
    return pl.pallas_call(
        _hupd_body,
        in_specs=[
            full((N, HID)), full((N, HID)), full((N, HID)), full((N, HID)),
            full((1, HID)), full((1, HID)),
        ],
        out_specs=full((N, HID)),
        out_shape=jax.ShapeDtypeStruct((N, HID), jnp.float32),
    )(h_in, ah, num, den, g, b)


def _mlp_body(h_ref, w1, b1, w2, b2, w3, b3, o_ref):
    y = h_ref[...]
    y = jnp.maximum(y @ w1[...] + b1[...], 0.0)
    y = jnp.maximum(y @ w2[...] + b2[...], 0.0)
    o_ref[...] = y @ w3[...] + b3[...]


def _mlp_readout(h, mlp):
    blk = 2000
    grid = (N // blk,)
    w1, b1 = mlp[0]["W"], mlp[0]["b"].reshape(1, -1)
    w2, b2 = mlp[1]["W"], mlp[1]["b"].reshape(1, -1)
    w3, b3 = mlp[2]["W"], mlp[2]["b"].reshape(1, -1)
    full = lambda s: pl.BlockSpec(s, lambda i: (0,) * len(s))
    return pl.pallas_call(
        _mlp_body,
        grid=grid,
        in_specs=[
            pl.BlockSpec((blk, HID), lambda i: (i, 0)),
            full(w1.shape), full(b1.shape),
            full(w2.shape), full(b2.shape),
            full(w3.shape), full(b3.shape),
        ],
        out_specs=pl.BlockSpec((blk, w3.shape[1]), lambda i: (i, 0)),
        out_shape=jax.ShapeDtypeStruct((N, w3.shape[1]), jnp.float32),
    )(h, w1, b1, w2, b2, w3, b3)


def kernel(h, edge_index, edge_feat, We, be, layers, mlp):
    src = edge_index[0].astype(jnp.int32)
    dst = edge_index[1].astype(jnp.int32)
    zeros = jnp.zeros((_RPS, 128), jnp.float32)

    e_prev = None
    e_new_prev = None
    e_stats_prev = None
    for li, lp in enumerate(layers):
        wcat = jnp.concatenate(
            [lp["WA"], lp["WD"], lp["WB"], lp["WE"]], axis=1)
        bcat = jnp.concatenate(
            [lp["bA"], lp["bD"], lp["bB"], lp["bE"]], axis=0).reshape(1, 512)
        ah, tsrc, tdst = _node_mm(h, wcat, bcat)

        if li == 0:
            e_cur, ce = _edge0(edge_feat, We, be.reshape(1, HID),
                               lp["WC"], lp["bC"].reshape(1, HID))
        else:
            s1, s2 = e_stats_prev
            mu = (s1[0:1] / E)
            var = s2[0:1] / E - mu * mu
            ivar = jax.lax.rsqrt(var + 1e-5)
            lpp = layers[li - 1]
            e_cur, ce = _edge_update(
                e_prev, e_new_prev, mu, ivar,
                lpp["bn_e_g"].reshape(1, HID), lpp["bn_e_b"].reshape(1, HID),
                lp["WC"], lp["bC"].reshape(1, HID))

        rows_src, rows_dst = _sc_gather2(tsrc, tdst, src, dst)
        e_new, sig, msg, s1, s2 = _gate(rows_src, rows_dst, ce)
        num, den = _sc_scatter(msg, sig, dst, zeros)
        h = _h_update(h, ah, num, den,
                      lp["bn_h_g"].reshape(1, HID), lp["bn_h_b"].reshape(1, HID))
        e_prev, e_new_prev, e_stats_prev = e_cur, e_new, (s1, s2)

    return _mlp_readout(h, mlp)


# speedup vs baseline: 1.5207x; 1.1082x over previous
import functools

import jax
import jax.numpy as jnp
from jax import lax
from jax.experimental import pallas as pl
from jax.experimental.pallas import tpu as pltpu
from jax.experimental.pallas import tpu_sc as plsc

N = 10000
E = 320000
HID = 128


_NC = 2
_NS = 16
_NW = _NC * _NS
_BLK = 128
_NBLK = E // _BLK
_SITER = -(-_NBLK // _NS)
_RPS = 632
_RPS_LAST = N - 15 * _RPS

_GBLK = 128
_GNBLK = E // _GBLK
_GITER = -(-_GNBLK // _NW)


def _sc_gather2(tsrc, tdst, src, dst):
    mesh = plsc.VectorSubcoreMesh(core_axis_name="c", subcore_axis_name="s")

    @functools.partial(
        pl.kernel,
        out_type=[
            jax.ShapeDtypeStruct((E, 128), jnp.int32),
            jax.ShapeDtypeStruct((E, 128), jnp.float32),
        ],
        mesh=mesh,
        scratch_types=[
            pltpu.VMEM((_GBLK,), jnp.int32), pltpu.VMEM((_GBLK,), jnp.int32),
            pltpu.VMEM((_GBLK,), jnp.int32), pltpu.VMEM((_GBLK,), jnp.int32),
            pltpu.VMEM((_GBLK, 128), jnp.int32),
            pltpu.VMEM((_GBLK, 128), jnp.int32),
            pltpu.VMEM((_GBLK, 128), jnp.float32),
            pltpu.VMEM((_GBLK, 128), jnp.float32),
            pltpu.SemaphoreType.DMA, pltpu.SemaphoreType.DMA,
            pltpu.SemaphoreType.DMA, pltpu.SemaphoreType.DMA,
            pltpu.SemaphoreType.DMA, pltpu.SemaphoreType.DMA,
            pltpu.SemaphoreType.DMA, pltpu.SemaphoreType.DMA,
        ],
    )
    def k(ts_hbm, td_hbm, src_hbm, dst_hbm, osrc_hbm, odst_hbm,
          si0, si1, di0, di1, A0, A1, E0, E1,
          sA0, sA1, sE0, sE1, sO0, sO1, sP0, sP1):
        wid = lax.axis_index("s") * _NC + lax.axis_index("c")

        def issue(b, si, di, A, Ebuf, sA, sE):
            base = b * _GBLK
            pltpu.sync_copy(src_hbm.at[pl.ds(base, _GBLK)], si)
            pltpu.sync_copy(dst_hbm.at[pl.ds(base, _GBLK)], di)
            pltpu.async_copy(ts_hbm.at[si], A, sA)
            pltpu.async_copy(td_hbm.at[di], Ebuf, sE)

        def half(t, cur, nxt):
            (csi, cdi, cA, cE, csA, csE, csO, csP) = cur
            (nsi, ndi, nA, nE, nsA, nsE, nsO, nsP) = nxt
            b = t * _NW + wid
            bn = b + _NW

            @pl.when(bn < _GNBLK)
            def _nxt():
                @pl.when(t >= 1)
                def _wo():
                    pltpu.make_async_copy(
                        nA, osrc_hbm.at[pl.ds(0, _GBLK)], nsO).wait()
                    pltpu.make_async_copy(
                        nE, odst_hbm.at[pl.ds(0, _GBLK)], nsP).wait()
                issue(bn, nsi, ndi, nA, nE, nsA, nsE)

            @pl.when(b < _GNBLK)
            def _cur():
                base = b * _GBLK
                pltpu.make_async_copy(ts_hbm.at[csi], cA, csA).wait()
                pltpu.make_async_copy(td_hbm.at[cdi], cE, csE).wait()
                pltpu.async_copy(cA, osrc_hbm.at[pl.ds(base, _GBLK)], csO)
                pltpu.async_copy(cE, odst_hbm.at[pl.ds(base, _GBLK)], csP)

        bufs0 = (si0, di0, A0, E0, sA0, sE0, sO0, sP0)
        bufs1 = (si1, di1, A1, E1, sA1, sE1, sO1, sP1)
        issue(wid, si0, di0, A0, E0, sA0, sE0)

        def body(t, carry):
            @pl.when(t % 2 == 0)
            def _e():
                half(t, bufs0, bufs1)

            @pl.when(t % 2 == 1)
            def _o():
                half(t, bufs1, bufs0)
            return carry

        lax.fori_loop(0, _GITER, body, 0)
        pltpu.make_async_copy(A0, osrc_hbm.at[pl.ds(0, _GBLK)], sO0).wait()
        pltpu.make_async_copy(E0, odst_hbm.at[pl.ds(0, _GBLK)], sP0).wait()
        pltpu.make_async_copy(A1, osrc_hbm.at[pl.ds(0, _GBLK)], sO1).wait()
        pltpu.make_async_copy(E1, odst_hbm.at[pl.ds(0, _GBLK)], sP1).wait()

    return k(tsrc, tdst, src, dst)


def _sc_scatter(msg, sig, dst, zeros):
    mesh = plsc.VectorSubcoreMesh(core_axis_name="c", subcore_axis_name="s")

    @functools.partial(
        pl.kernel,
        out_type=[
            jax.ShapeDtypeStruct((N, 128), jnp.float32),
            jax.ShapeDtypeStruct((N, 128), jnp.float32),
        ],
        mesh=mesh,
        scratch_types=[
            pltpu.VMEM((_BLK,), jnp.int32),
            pltpu.VMEM((_BLK,), jnp.int32),
            pltpu.VMEM((_BLK, 128), jnp.float32),
            pltpu.VMEM((_BLK, 128), jnp.float32),
            pltpu.VMEM_SHARED((N, 128), jnp.float32),
            pltpu.SemaphoreType.DMA, pltpu.SemaphoreType.DMA,
            pltpu.SemaphoreType.DMA, pltpu.SemaphoreType.DMA,
        ],
    )
    def k(msg_hbm, sig_hbm, dst_hbm, z_hbm, num_hbm, den_hbm,
          idx0, idx1, upd0, upd1, acc, sI0, sI1, sU0, sU1):
        c = lax.axis_index("c")
        s = lax.axis_index("s")

        @pl.when(s < 15)
        def _z0():
            pltpu.sync_copy(z_hbm, acc.at[pl.ds(s * _RPS, _RPS)])

        @pl.when(s == 15)
        def _z1():
            pltpu.sync_copy(z_hbm.at[pl.ds(0, _RPS_LAST)],
                            acc.at[pl.ds(15 * _RPS, _RPS_LAST)])

        plsc.subcore_barrier()

        def issue(b, idx_v, upd_v, sI, sU):
            base = b * _BLK
            pltpu.async_copy(dst_hbm.at[pl.ds(base, _BLK)], idx_v, sI)

            @pl.when(c == 0)
            def _c0():
                pltpu.async_copy(msg_hbm.at[pl.ds(base, _BLK)], upd_v, sU)

            @pl.when(c == 1)
            def _c1():
                pltpu.async_copy(sig_hbm.at[pl.ds(base, _BLK)], upd_v, sU)

        def half(t, cur, nxt):
            (cidx, cupd, csI, csU) = cur
            (nidx, nupd, nsI, nsU) = nxt
            b = t * _NS + s
            bn = b + _NS

            @pl.when(bn < _NBLK)
            def _nxt():
                issue(bn, nidx, nupd, nsI, nsU)

            @pl.when(b < _NBLK)
            def _cur():
                pltpu.make_async_copy(
                    dst_hbm.at[pl.ds(0, _BLK)], cidx, csI).wait()
                pltpu.make_async_copy(
                    msg_hbm.at[pl.ds(0, _BLK)], cupd, csU).wait()
                pltpu.sync_copy(cupd, acc.at[cidx], add=True)

        bufs0 = (idx0, upd0, sI0, sU0)
        bufs1 = (idx1, upd1, sI1, sU1)
        issue(s, idx0, upd0, sI0, sU0)

        def body(t, carry):
            @pl.when(t % 2 == 0)
            def _e():
                half(t, bufs0, bufs1)

            @pl.when(t % 2 == 1)
            def _o():
                half(t, bufs1, bufs0)
            return carry

        lax.fori_loop(0, _SITER, body, 0)
        plsc.subcore_barrier()

        @pl.when(jnp.logical_and(c == 0, s < 15))
        def _w00():
            pltpu.sync_copy(acc.at[pl.ds(s * _RPS, _RPS)],
                            num_hbm.at[pl.ds(s * _RPS, _RPS)])

        @pl.when(jnp.logical_and(c == 0, s == 15))
        def _w01():
            pltpu.sync_copy(acc.at[pl.ds(15 * _RPS, _RPS_LAST)],
                            num_hbm.at[pl.ds(15 * _RPS, _RPS_LAST)])

        @pl.when(jnp.logical_and(c == 1, s < 15))
        def _w10():
            pltpu.sync_copy(acc.at[pl.ds(s * _RPS, _RPS)],
                            den_hbm.at[pl.ds(s * _RPS, _RPS)])

        @pl.when(jnp.logical_and(c == 1, s == 15))
        def _w11():
            pltpu.sync_copy(acc.at[pl.ds(15 * _RPS, _RPS_LAST)],
                            den_hbm.at[pl.ds(15 * _RPS, _RPS_LAST)])

    return k(msg, sig, dst, zeros)


_NBK = 2000
_EBK = 1280


def _node_mm_body(h_ref, w_ref, b_ref, ah_ref, ts_ref, td_ref):
    x = h_ref[...] @ w_ref[...] + b_ref[...]
    ah_ref[...] = x[:, 0:128]
    di = jax.lax.bitcast_convert_type(x[:, 128:256], jnp.uint32)
    bi = jax.lax.bitcast_convert_type(x[:, 256:384], jnp.uint32)
    packed = ((di + jnp.uint32(0x8000)) >> 16) | (
        (bi + jnp.uint32(0x8000)) & jnp.uint32(0xFFFF0000))
    ts_ref[...] = jax.lax.bitcast_convert_type(packed, jnp.int32)
    td_ref[...] = x[:, 384:512]


def _node_mm(h, wcat, bcat):
    grid = (N // _NBK,)
    return pl.pallas_call(
        _node_mm_body,
        grid=grid,
        in_specs=[
            pl.BlockSpec((_NBK, HID), lambda i: (i, 0)),
            pl.BlockSpec((HID, 512), lambda i: (0, 0)),
            pl.BlockSpec((1, 512), lambda i: (0, 0)),
        ],
        out_specs=[
            pl.BlockSpec((_NBK, 128), lambda i: (i, 0)),
            pl.BlockSpec((_NBK, 128), lambda i: (i, 0)),
            pl.BlockSpec((_NBK, 128), lambda i: (i, 0)),
        ],
        out_shape=[
            jax.ShapeDtypeStruct((N, 128), jnp.float32),
            jax.ShapeDtypeStruct((N, 128), jnp.int32),
            jax.ShapeDtypeStruct((N, 128), jnp.float32),
        ],
    )(h, wcat, bcat)


def _edge0_body(ef_ref, we_ref, be_ref, wc_ref, bc_ref, e_ref, ce_ref):
    e = ef_ref[...] @ we_ref[...] + be_ref[...]
    e_ref[...] = e
    ce_ref[...] = e @ wc_ref[...] + bc_ref[...]


def _edge0(edge_feat, We, be, wc, bc):
    grid = (E // _EBK,)
    d_in = edge_feat.shape[1]
    return pl.pallas_call(
        _edge0_body,
        grid=grid,
        in_specs=[
            pl.BlockSpec((_EBK, d_in), lambda i: (i, 0)),
            pl.BlockSpec((d_in, HID), lambda i: (0, 0)),
            pl.BlockSpec((1, HID), lambda i: (0, 0)),
            pl.BlockSpec((HID, HID), lambda i: (0, 0)),
            pl.BlockSpec((1, HID), lambda i: (0, 0)),
        ],
        out_specs=[
            pl.BlockSpec((_EBK, HID), lambda i: (i, 0)),
            pl.BlockSpec((_EBK, HID), lambda i: (i, 0)),
        ],
        out_shape=[
            jax.ShapeDtypeStruct((E, HID), jnp.float32),
            jax.ShapeDtypeStruct((E, HID), jnp.float32),
        ],
    )(edge_feat, We, be, wc, bc)


def _eupd_body(ep_ref, en_ref, mu_ref, iv_ref, g_ref, b_ref,
               wc_ref, bc_ref, e_ref, ce_ref):
    bn = (en_ref[...] - mu_ref[...]) * iv_ref[...] * g_ref[...] + b_ref[...]
    e = ep_ref[...] + jnp.maximum(bn, 0.0)
    e_ref[...] = e
    ce_ref[...] = e @ wc_ref[...] + bc_ref[...]


def _edge_update(e_prev, e_new_prev, mu, ivar, g, b, wc, bc):
    grid = (E // _EBK,)
    row = lambda: pl.BlockSpec((1, HID), lambda i: (0, 0))
    return pl.pallas_call(
        _eupd_body,
        grid=grid,
        in_specs=[
            pl.BlockSpec((_EBK, HID), lambda i: (i, 0)),
            pl.BlockSpec((_EBK, HID), lambda i: (i, 0)),
            row(), row(), row(), row(),
            pl.BlockSpec((HID, HID), lambda i: (0, 0)),
            row(),
        ],
        out_specs=[
            pl.BlockSpec((_EBK, HID), lambda i: (i, 0)),
            pl.BlockSpec((_EBK, HID), lambda i: (i, 0)),
        ],
        out_shape=[
            jax.ShapeDtypeStruct((E, HID), jnp.float32),
            jax.ShapeDtypeStruct((E, HID), jnp.float32),
        ],
    )(e_prev, e_new_prev, mu, ivar, g, b, wc, bc)


def _gate_body(rs_ref, rd_ref, ce_ref, en_ref, sig_ref, msg_ref,
               s1_ref, s2_ref):
    i = pl.program_id(0)
    w = jax.lax.bitcast_convert_type(rs_ref[...], jnp.uint32)
    dh = jax.lax.bitcast_convert_type(w << 16, jnp.float32)
    bh = jax.lax.bitcast_convert_type(w & jnp.uint32(0xFFFF0000), jnp.float32)
    e_new = dh + rd_ref[...] + ce_ref[...]
    sig = jax.nn.sigmoid(e_new)
    en_ref[...] = e_new
    sig_ref[...] = sig
    msg_ref[...] = bh * sig
    ps = jnp.sum(e_new, axis=0, keepdims=True)
    pq = jnp.sum(e_new * e_new, axis=0, keepdims=True)

    @pl.when(i == 0)
    def _():
        s1_ref[...] = jnp.zeros_like(s1_ref)
        s2_ref[...] = jnp.zeros_like(s2_ref)

    s1_ref[0:1, :] += ps
    s2_ref[0:1, :] += pq


def _gate(rows_src, rows_dst, ce):
    grid = (E // _EBK,)
    return pl.pallas_call(
        _gate_body,
        grid=grid,
        in_specs=[
            pl.BlockSpec((_EBK, 128), lambda i: (i, 0)),
            pl.BlockSpec((_EBK, 128), lambda i: (i, 0)),
            pl.BlockSpec((_EBK, 128), lambda i: (i, 0)),
        ],
        out_specs=[
            pl.BlockSpec((_EBK, 128), lambda i: (i, 0)),
            pl.BlockSpec((_EBK, 128), lambda i: (i, 0)),
            pl.BlockSpec((_EBK, 128), lambda i: (i, 0)),
            pl.BlockSpec((8, 128), lambda i: (0, 0)),
            pl.BlockSpec((8, 128), lambda i: (0, 0)),
        ],
        out_shape=[
            jax.ShapeDtypeStruct((E, 128), jnp.float32),
            jax.ShapeDtypeStruct((E, 128), jnp.float32),
            jax.ShapeDtypeStruct((E, 128), jnp.float32),
            jax.ShapeDtypeStruct((8, 128), jnp.float32),
            jax.ShapeDtypeStruct((8, 128), jnp.float32),
        ],
    )(rows_src, rows_dst, ce)


def _hupd_body(hin_ref, ah_ref, num_ref, den_ref, g_ref, b_ref, out_ref):
    h_new = ah_ref[...] + num_ref[...] / (den_ref[...] + 1e-6)
    mu = jnp.mean(h_new, axis=0, keepdims=True)
    var = jnp.mean((h_new - mu) ** 2, axis=0, keepdims=True)
    bn = (h_new - mu) * jax.lax.rsqrt(var + 1e-5) * g_ref[...] + b_ref[...]
    out_ref[...] = hin_ref[...] + jnp.maximum(bn, 0.0)


def _h_update(h_in, ah, num, den, g, b):
    full = lambda s: pl.BlockSpec(s, lambda: tuple(0 for _ in s))
    return pl.pallas_call(
        _hupd_body,
        in_specs=[
            full((N, HID)), full((N, HID)), full((N, HID)), full((N, HID)),
            full((1, HID)), full((1, HID)),
        ],
        out_specs=full((N, HID)),
        out_shape=jax.ShapeDtypeStruct((N, HID), jnp.float32),
    )(h_in, ah, num, den, g, b)


def _mlp_body(h_ref, w1, b1, w2, b2, w3, b3, o_ref):
    y = h_ref[...]
    y = jnp.maximum(y @ w1[...] + b1[...], 0.0)
    y = jnp.maximum(y @ w2[...] + b2[...], 0.0)
    o_ref[...] = y @ w3[...] + b3[...]


def _mlp_readout(h, mlp):
    blk = 2000
    grid = (N // blk,)
    w1, b1 = mlp[0]["W"], mlp[0]["b"].reshape(1, -1)
    w2, b2 = mlp[1]["W"], mlp[1]["b"].reshape(1, -1)
    w3, b3 = mlp[2]["W"], mlp[2]["b"].reshape(1, -1)
    full = lambda s: pl.BlockSpec(s, lambda i: (0,) * len(s))
    return pl.pallas_call(
        _mlp_body,
        grid=grid,
        in_specs=[
            pl.BlockSpec((blk, HID), lambda i: (i, 0)),
            full(w1.shape), full(b1.shape),
            full(w2.shape), full(b2.shape),
            full(w3.shape), full(b3.shape),
        ],
        out_specs=pl.BlockSpec((blk, w3.shape[1]), lambda i: (i, 0)),
        out_shape=jax.ShapeDtypeStruct((N, w3.shape[1]), jnp.float32),
    )(h, w1, b1, w2, b2, w3, b3)


def kernel(h, edge_index, edge_feat, We, be, layers, mlp):
    src = edge_index[0].astype(jnp.int32)
    dst = edge_index[1].astype(jnp.int32)
    zeros = jnp.zeros((_RPS, 128), jnp.float32)

    e_prev = None
    e_new_prev = None
    e_stats_prev = None
    for li, lp in enumerate(layers):
        wcat = jnp.concatenate(
            [lp["WA"], lp["WD"], lp["WB"], lp["WE"]], axis=1)
        bcat = jnp.concatenate(
            [lp["bA"], lp["bD"], lp["bB"], lp["bE"]], axis=0).reshape(1, 512)
        ah, tsrc, tdst = _node_mm(h, wcat, bcat)

        if li == 0:
            e_cur, ce = _edge0(edge_feat, We, be.reshape(1, HID),
                               lp["WC"], lp["bC"].reshape(1, HID))
        else:
            s1, s2 = e_stats_prev
            mu = (s1[0:1] / E)
            var = s2[0:1] / E - mu * mu
            ivar = jax.lax.rsqrt(var + 1e-5)
            lpp = layers[li - 1]
            e_cur, ce = _edge_update(
                e_prev, e_new_prev, mu, ivar,
                lpp["bn_e_g"].reshape(1, HID), lpp["bn_e_b"].reshape(1, HID),
                lp["WC"], lp["bC"].reshape(1, HID))

        rows_src, rows_dst = _sc_gather2(tsrc, tdst, src, dst)
        e_new, sig, msg, s1, s2 = _gate(rows_src, rows_dst, ce)
        num, den = _sc_scatter(msg, sig, dst, zeros)
        h = _h_update(h, ah, num, den,
                      lp["bn_h_g"].reshape(1, HID), lp["bn_h_b"].reshape(1, HID))
        e_prev, e_new_prev, e_stats_prev = e_cur, e_new, (s1, s2)

    return _mlp_readout(h, mlp)
